# baseline (speedup 1.0000x reference)
import functools

import jax
import jax.numpy as jnp
from jax.experimental import pallas as pl
from jax.experimental.pallas import tpu as pltpu

_B = 2
_NP = 2000
_NG = 100
_H = 256
_W = 256
_TRAIN_ROIS = 200
_POS = 66
_NEG = 134
_MH = 28
_MW = 28
_BBOX_STD = (0.1, 0.1, 0.2, 0.2)


def _topk_loop(scores, k):
    n = scores.shape[0]
    row_iota = jax.lax.broadcasted_iota(jnp.int32, (n, 1), 0)
    k_iota = jax.lax.broadcasted_iota(jnp.int32, (k, 1), 0)

    def body(i, carry):
        sc, vals, idxs = carry
        m = jnp.max(sc)
        a = jnp.min(jnp.where(sc == m, row_iota, n)).astype(jnp.int32)
        vals = jnp.where(k_iota == i, m, vals)
        idxs = jnp.where(k_iota == i, a, idxs)
        sc = jnp.where(row_iota == a, -jnp.inf, sc)
        return sc, vals, idxs

    vals0 = jnp.zeros((k, 1), jnp.float32)
    idxs0 = jnp.zeros((k, 1), jnp.int32)
    _, vals, idxs = jax.lax.fori_loop(0, k, body, (scores, vals0, idxs0))
    return vals, idxs


def _onehot(idxs, n):
    k = idxs.shape[0]
    cols = jax.lax.broadcasted_iota(jnp.int32, (k, n), 1)
    return (cols == idxs).astype(jnp.float32)


def _select_kernel(props_ref, gtb_ref, gtc_ref,
                   rois_ref, cls_ref, deltas_ref,
                   assign_ref, valid_ref, praw_ref):
    props = props_ref[0]
    gtb = gtb_ref[0]
    gtc = gtc_ref[0].astype(jnp.float32)

    gbt = gtb.T
    py1, px1, py2, px2 = (props[:, 0:1], props[:, 1:2],
                          props[:, 2:3], props[:, 3:4])
    gy1, gx1, gy2, gx2 = (gbt[0:1, :], gbt[1:2, :], gbt[2:3, :], gbt[3:4, :])
    iy1 = jnp.maximum(py1, gy1)
    ix1 = jnp.maximum(px1, gx1)
    iy2 = jnp.minimum(py2, gy2)
    ix2 = jnp.minimum(px2, gx2)
    inter = jnp.maximum(iy2 - iy1, 0.0) * jnp.maximum(ix2 - ix1, 0.0)
    area_p = (py2 - py1) * (px2 - px1)
    area_g = (gy2 - gy1) * (gx2 - gx1)
    overlaps = inter / (area_p + area_g - inter)

    roi_iou_max = jnp.max(overlaps, axis=1, keepdims=True)

    pos_scores = jnp.where(roi_iou_max >= 0.5, roi_iou_max, -1.0)
    pos_vals, pos_idx = _topk_loop(pos_scores, _POS)
    pos_valid = (pos_vals >= 0.5).astype(jnp.float32)

    neg_scores = jnp.where(roi_iou_max < 0.5, 1.0 - roi_iou_max, -jnp.inf)
    neg_vals, neg_idx = _topk_loop(neg_scores, _NEG)
    neg_valid = (neg_vals > -jnp.inf).astype(jnp.float32)

    oh_pos = _onehot(pos_idx, _NP)
    oh_neg = _onehot(neg_idx, _NP)
    pos_rois = jnp.dot(oh_pos, props,
                       preferred_element_type=jnp.float32, precision=jax.lax.Precision.HIGHEST)
    neg_rois = jnp.dot(oh_neg, props,
                       preferred_element_type=jnp.float32, precision=jax.lax.Precision.HIGHEST) * neg_valid
    pos_ov = jnp.dot(oh_pos, overlaps,
                     preferred_element_type=jnp.float32, precision=jax.lax.Precision.HIGHEST)
    ov_max = jnp.max(pos_ov, axis=1, keepdims=True)
    ng_iota = jax.lax.broadcasted_iota(jnp.int32, (_POS, _NG), 1)
    assign = jnp.min(jnp.where(pos_ov == ov_max, ng_iota, _NG),
                     axis=1, keepdims=True).astype(jnp.int32)
    oh_as = _onehot(assign, _NG)
    roi_gt = jnp.dot(oh_as, gtb,
                     preferred_element_type=jnp.float32, precision=jax.lax.Precision.HIGHEST)
    roi_cls = jnp.dot(oh_as, gtc,
                      preferred_element_type=jnp.float32, precision=jax.lax.Precision.HIGHEST)
    roi_cls = roi_cls * pos_valid

    h = pos_rois[:, 2:3] - pos_rois[:, 0:1]
    w = pos_rois[:, 3:4] - pos_rois[:, 1:2]
    cy = pos_rois[:, 0:1] + 0.5 * h
    cx = pos_rois[:, 1:2] + 0.5 * w
    gh = roi_gt[:, 2:3] - roi_gt[:, 0:1]
    gw = roi_gt[:, 3:4] - roi_gt[:, 1:2]
    gcy = roi_gt[:, 0:1] + 0.5 * gh
    gcx = roi_gt[:, 1:2] + 0.5 * gw
    dy = (gcy - cy) / h / _BBOX_STD[0]
    dx = (gcx - cx) / w / _BBOX_STD[1]
    dh = jnp.log(gh / h) / _BBOX_STD[2]
    dw = jnp.log(gw / w) / _BBOX_STD[3]
    deltas = jnp.concatenate([dy, dx, dh, dw], axis=1) * pos_valid

    rois = jnp.concatenate([pos_rois * pos_valid, neg_rois], axis=0)
    cls_full = jnp.concatenate(
        [roi_cls, jnp.zeros((_NEG, 1), jnp.float32)], axis=0)
    deltas_full = jnp.concatenate(
        [deltas, jnp.zeros((_NEG, 4), jnp.float32)], axis=0)

    rois_ref[0] = rois
    cls_ref[0] = (cls_full + 0.5).astype(jnp.int32)
    deltas_ref[0] = deltas_full
    assign_ref[0] = assign
    valid_ref[0] = pos_valid
    praw_ref[0] = pos_rois


def _crop_kernel(assign_sref, masks_ref, boxes_ref, out_ref):
    p = pl.program_id(1)
    sel = (jax.lax.broadcasted_iota(jnp.int32, (1, _POS), 1)
           == p).astype(jnp.float32)
    box = jnp.dot(sel, boxes_ref[0],
                  preferred_element_type=jnp.float32, precision=jax.lax.Precision.HIGHEST)
    y1, x1, y2, x2 = box[0, 0], box[0, 1], box[0, 2], box[0, 3]

    iy = (jax.lax.broadcasted_iota(jnp.int32, (_MH, 1), 0)
          .astype(jnp.float32) / (_MH - 1))
    ix = (jax.lax.broadcasted_iota(jnp.int32, (1, _MW), 1)
          .astype(jnp.float32) / (_MW - 1))
    ys = (y1 + iy * (y2 - y1)) * (_H - 1)
    xs = (x1 + ix * (x2 - x1)) * (_W - 1)

    y0 = jnp.floor(ys)
    x0 = jnp.floor(xs)
    y0i = jnp.clip(y0.astype(jnp.int32), 0, _H - 1)
    y1i = jnp.clip(y0i + 1, 0, _H - 1)
    x0i = jnp.clip(x0.astype(jnp.int32), 0, _W - 1)
    x1i = jnp.clip(x0i + 1, 0, _W - 1)
    wy = ys - y0
    wx = xs - x0

    hcols = jax.lax.broadcasted_iota(jnp.int32, (_MH, _H), 1)
    wrows = jax.lax.broadcasted_iota(jnp.int32, (_W, _MW), 0)
    Wy = ((hcols == y0i).astype(jnp.float32) * (1.0 - wy)
          + (hcols == y1i).astype(jnp.float32) * wy)
    Wx = ((wrows == x0i).astype(jnp.float32) * (1.0 - wx)
          + (wrows == x1i).astype(jnp.float32) * wx)

    img = masks_ref[0, 0]
    crop = jnp.dot(jnp.dot(Wy, img, preferred_element_type=jnp.float32, precision=jax.lax.Precision.HIGHEST),
                   Wx, preferred_element_type=jnp.float32, precision=jax.lax.Precision.HIGHEST)
    out_ref[0, 0] = jnp.round(crop)


@jax.jit
def kernel(proposals, prior_class_ids, prior_boxes, prior_masks):
    gtc = prior_class_ids.reshape(_B, _NG, 1)
    sel_out = pl.pallas_call(
        _select_kernel,
        grid=(_B,),
        in_specs=[
            pl.BlockSpec((1, _NP, 4), lambda b: (b, 0, 0)),
            pl.BlockSpec((1, _NG, 4), lambda b: (b, 0, 0)),
            pl.BlockSpec((1, _NG, 1), lambda b: (b, 0, 0)),
        ],
        out_specs=[
            pl.BlockSpec((1, _TRAIN_ROIS, 4), lambda b: (b, 0, 0)),
            pl.BlockSpec((1, _TRAIN_ROIS, 1), lambda b: (b, 0, 0)),
            pl.BlockSpec((1, _TRAIN_ROIS, 4), lambda b: (b, 0, 0)),
            pl.BlockSpec((1, _POS, 1), lambda b: (b, 0, 0)),
            pl.BlockSpec((1, _POS, 1), lambda b: (b, 0, 0)),
            pl.BlockSpec((1, _POS, 4), lambda b: (b, 0, 0)),
        ],
        out_shape=[
            jax.ShapeDtypeStruct((_B, _TRAIN_ROIS, 4), jnp.float32),
            jax.ShapeDtypeStruct((_B, _TRAIN_ROIS, 1), jnp.int32),
            jax.ShapeDtypeStruct((_B, _TRAIN_ROIS, 4), jnp.float32),
            jax.ShapeDtypeStruct((_B, _POS, 1), jnp.int32),
            jax.ShapeDtypeStruct((_B, _POS, 1), jnp.float32),
            jax.ShapeDtypeStruct((_B, _POS, 4), jnp.float32),
        ],
    )(proposals, prior_boxes, gtc)
    rois, cls, deltas, assign, valid, praw = sel_out

    masks_t = jnp.transpose(prior_masks, (0, 3, 1, 2))
    assign2 = assign.reshape(_B, _POS)
    valid2 = valid.reshape(_B, _POS, 1)

    crop = pl.pallas_call(
        _crop_kernel,
        grid_spec=pltpu.PrefetchScalarGridSpec(
            num_scalar_prefetch=1,
            grid=(_B, _POS),
            in_specs=[
                pl.BlockSpec((1, 1, _H, _W),
                             lambda b, p, a: (b, a[b, p], 0, 0)),
                pl.BlockSpec((1, _POS, 4), lambda b, p, a: (b, 0, 0)),
            ],
            out_specs=pl.BlockSpec((1, 1, _MH, _MW),
                                   lambda b, p, a: (b, p, 0, 0)),
        ),
        out_shape=jax.ShapeDtypeStruct((_B, _POS, _MH, _MW), jnp.float32),
    )(assign2, masks_t, praw)

    crop = crop * valid2[:, :, :, None]
    masks_full = jnp.concatenate(
        [crop, jnp.zeros((_B, _NEG, _MH, _MW), jnp.float32)], axis=1)
    return rois, cls.reshape(_B, _TRAIN_ROIS), deltas, masks_full
